# Initial kernel scaffold; baseline (speedup 1.0000x reference)
import jax
import jax.numpy as jnp
from jax.experimental import pallas as pl


def kernel(task_id, bsz, taskID_embed, in_proj_weight, in_proj_bias, out_proj_weight, out_proj_bias, expert_keys, fc_gate_w, fc_gate_b, fc_noise_w, fc_noise_b, noise):
    raise NotImplementedError("write your pallas kernel here")


# speedup vs baseline: 6.0528x; 6.0528x over previous
import functools

import jax
import jax.numpy as jnp
from jax.experimental import pallas as pl
from jax.experimental.pallas import tpu as pltpu

E_DIM = 32
N_HEADS = 4
HEAD_DIM = E_DIM // N_HEADS
NUM_EXPERTS = 64
NOISE_EPS = 0.01
BLK = 1024


def _routing_body(tid_ref, emb_ref, ipw_ref, ipb_ref, ek_ref, fgw_ref,
                  fgb_ref, fnw_ref, fnb_ref, noise_ref, gates_ref, load_ref,
                  cs_ref):
    step = pl.program_id(0)

    @pl.when(step == 0)
    def _prologue():
        tid = tid_ref[...].reshape(1, 1)
        row_ids = jax.lax.broadcasted_iota(jnp.int32, (6, 1), 0)
        e_rows = jnp.where(row_ids == tid, emb_ref[...], 0.0)
        e = jnp.sum(e_rows, axis=0, keepdims=True)

        wq = ipw_ref[0:E_DIM, :]
        wk = ipw_ref[E_DIM:2 * E_DIM, :]
        bq = ipb_ref[0, 0:E_DIM]
        bk = ipb_ref[0, E_DIM:2 * E_DIM]

        dn = (((1,), (1,)), ((), ()))
        q = jax.lax.dot_general(e, wq, dn,
                                preferred_element_type=jnp.float32) + bq[None, :]
        k = jax.lax.dot_general(ek_ref[...], wk, dn,
                                preferred_element_type=jnp.float32) + bk[None, :]

        s_full = k * q
        d_ids = jax.lax.broadcasted_iota(jnp.int32, (E_DIM, N_HEADS), 0)
        h_ids = jax.lax.broadcasted_iota(jnp.int32, (E_DIM, N_HEADS), 1)
        head_mask = ((d_ids // HEAD_DIM) == h_ids).astype(jnp.float32)
        dn0 = (((1,), (0,)), ((), ()))
        scores = jax.lax.dot_general(s_full, head_mask, dn0,
                                     preferred_element_type=jnp.float32)
        scores = scores / jnp.sqrt(jnp.float32(HEAD_DIM))

        attn = jax.nn.softmax(scores, axis=0)
        avg = jnp.mean(attn, axis=1, keepdims=True)
        w = jax.nn.softmax(avg, axis=0).reshape(1, E_DIM)

        clean = jax.lax.dot_general(w, fgw_ref[...], dn,
                                    preferred_element_type=jnp.float32)
        clean = clean + fgb_ref[...]
        raw = jax.lax.dot_general(w, fnw_ref[...], dn,
                                  preferred_element_type=jnp.float32)
        raw = raw + fnb_ref[...]
        std = jnp.maximum(raw, 0.0) + jnp.log1p(jnp.exp(-jnp.abs(raw)))
        cs_ref[0:1, :] = clean
        cs_ref[1:2, :] = std + NOISE_EPS

    clean = cs_ref[0:1, :]
    std = cs_ref[1:2, :]
    logits = clean + noise_ref[...] * std

    lane = jax.lax.broadcasted_iota(jnp.int32, (BLK, NUM_EXPERTS), 1)
    m0 = jnp.max(logits, axis=1, keepdims=True)
    i0 = jnp.min(jnp.where(logits == m0, lane, NUM_EXPERTS), axis=1,
                 keepdims=True)
    masked = jnp.where(lane == i0, -jnp.inf, logits)
    m1 = jnp.max(masked, axis=1, keepdims=True)
    i1 = jnp.min(jnp.where(masked == m1, lane, NUM_EXPERTS), axis=1,
                 keepdims=True)

    e1 = jnp.exp(m1 - m0)
    g0 = 1.0 / (1.0 + e1)
    g1 = e1 / (1.0 + e1)
    gates = jnp.where(lane == i0, g0, jnp.where(lane == i1, g1, 0.0))
    gates_ref[...] = gates

    @pl.when(step == 0)
    def _init_load():
        load_ref[...] = jnp.zeros_like(load_ref)

    load_ref[...] += jnp.sum(gates, axis=0, keepdims=True)


def kernel(task_id, bsz, taskID_embed, in_proj_weight, in_proj_bias,
           out_proj_weight, out_proj_bias, expert_keys,
           fc_gate_w, fc_gate_b, fc_noise_w, fc_noise_b, noise):
    del bsz, out_proj_weight, out_proj_bias
    B = noise.shape[0]
    n_blk = B // BLK
    tid = jnp.asarray(task_id, jnp.int32).reshape(1, 1)
    ipb = in_proj_bias.reshape(1, -1)
    fgb = fc_gate_b.reshape(1, -1)
    fnb = fc_noise_b.reshape(1, -1)

    full = lambda shape: pl.BlockSpec(shape, lambda i: (0,) * len(shape))
    gates, load = pl.pallas_call(
        _routing_body,
        grid=(n_blk,),
        in_specs=[
            full((1, 1)),
            full((6, E_DIM)),
            full((3 * E_DIM, E_DIM)),
            full((1, 3 * E_DIM)),
            full((E_DIM, E_DIM)),
            full((NUM_EXPERTS, E_DIM)),
            full((1, NUM_EXPERTS)),
            full((NUM_EXPERTS, E_DIM)),
            full((1, NUM_EXPERTS)),
            pl.BlockSpec((BLK, NUM_EXPERTS), lambda i: (i, 0)),
        ],
        out_specs=[
            pl.BlockSpec((BLK, NUM_EXPERTS), lambda i: (i, 0)),
            full((1, NUM_EXPERTS)),
        ],
        out_shape=[
            jax.ShapeDtypeStruct((B, NUM_EXPERTS), jnp.float32),
            jax.ShapeDtypeStruct((1, NUM_EXPERTS), jnp.float32),
        ],
        scratch_shapes=[pltpu.VMEM((2, NUM_EXPERTS), jnp.float32)],
        compiler_params=pltpu.CompilerParams(
            dimension_semantics=("arbitrary",)),
    )(tid, taskID_embed, in_proj_weight, ipb, expert_keys,
      fc_gate_w, fgb, fc_noise_w, fnb, noise)
    return gates, load.reshape(NUM_EXPERTS)
